# baseline (speedup 1.0000x reference)
import functools

import jax
import jax.numpy as jnp
from jax import lax
from jax.experimental import pallas as pl
from jax.experimental.pallas import tpu as pltpu
from jax.experimental.pallas import tpu_sc as plsc

N = 10000
NFEAT = 128
NHID = 128
NH = N // 2

BM = 200
BK = 200

R_SC = 3200
N_TC = N - R_SC
NW = 32
RPW = R_SC // NW
NCH = N // 16

NB1 = N // BM
NB2 = N_TC // BM
NBK = NH // BK


def _pass1_body(adj_ref, x_ref, w1_ref, b1_ref, w2_ref, v_ref, s1_ref):
    i = pl.program_id(0)

    @pl.when(i == 0)
    def _():
        s1_ref[...] = jnp.dot(x_ref[...], w1_ref[...],
                              preferred_element_type=jnp.float32)

    h = jnp.dot(adj_ref[...], s1_ref[...],
                preferred_element_type=jnp.float32)
    h = jnp.maximum(h + b1_ref[...], 0.0)
    v_ref[...] = jnp.dot(h, w2_ref[...],
                         preferred_element_type=jnp.float32)


_sc_mesh = plsc.VectorSubcoreMesh(core_axis_name="c", subcore_axis_name="s")


@functools.partial(
    pl.kernel,
    mesh=_sc_mesh,
    out_type=jax.ShapeDtypeStruct((R_SC * 16,), jnp.float32),
    scratch_types=[
        pltpu.VMEM((N,), jnp.float32),
        pltpu.VMEM((N,), jnp.float32),
        pltpu.VMEM((N,), jnp.float32),
        pltpu.VMEM((RPW * 16,), jnp.float32),
        pltpu.SemaphoreType.DMA,
        pltpu.SemaphoreType.DMA,
    ],
)
def _sc_pass2(m_hbm, u_hbm, out_hbm, u_v, r0_v, r1_v, p_v, sem0, sem1):
    wid = lax.axis_index("s") * 2 + lax.axis_index("c")
    base = N_TC + wid * RPW
    pltpu.sync_copy(u_hbm, u_v)

    def row_dot(row_ref):
        def chunk(j, acc):
            return acc + row_ref[pl.ds(j * 16, 16)] * u_v[pl.ds(j * 16, 16)]
        return lax.fori_loop(0, NCH, chunk, jnp.zeros((16,), jnp.float32))

    pltpu.async_copy(m_hbm.at[base], r0_v, sem0)

    def body(k, carry):
        r0 = base + 2 * k
        pltpu.async_copy(m_hbm.at[r0 + 1], r1_v, sem1)
        pltpu.make_async_copy(m_hbm.at[r0], r0_v, sem0).wait()
        p_v[pl.ds((2 * k) * 16, 16)] = row_dot(r0_v)
        nxt = jnp.minimum(r0 + 2, N - 1)
        pltpu.async_copy(m_hbm.at[nxt], r0_v, sem0)
        pltpu.make_async_copy(m_hbm.at[r0 + 1], r1_v, sem1).wait()
        p_v[pl.ds((2 * k + 1) * 16, 16)] = row_dot(r1_v)
        return carry

    lax.fori_loop(0, RPW // 2, body, 0)
    pltpu.make_async_copy(m_hbm.at[base], r0_v, sem0).wait()
    pltpu.sync_copy(p_v, out_hbm.at[pl.ds(wid * RPW * 16, RPW * 16)])


def _pass2_body(adj_ref, v_ref, b2_ref, z_ref):
    z_ref[...] = (jnp.dot(adj_ref[...], v_ref[...],
                          preferred_element_type=jnp.float32)
                  + b2_ref[...])


def _readout_body(l1w_ref, ztc_ref, zp_ref, l1b_ref, l2w_ref, b2_ref,
                  l2b_ref, o_ref, z_ref):
    k = pl.program_id(0)

    @pl.when(k == 0)
    def _():
        z_ref[:N_TC, :] = ztc_ref[...]
        z_ref[N_TC:, :] = (jnp.sum(zp_ref[...], axis=1, keepdims=True)
                           + b2_ref[...])

    h = jnp.dot(l1w_ref[...], z_ref[...],
                preferred_element_type=jnp.float32)
    h = jnp.maximum(h + l1b_ref[...], 0.0)
    part = jnp.sum(h * l2w_ref[...]).reshape(1, 1)

    @pl.when(k == 0)
    def _():
        o_ref[...] = part + l2b_ref[...]

    @pl.when(k > 0)
    def _():
        o_ref[...] += part


def kernel(x, adj, W1, b1, W2, b2, L1_W, L1_b, L2_W, L2_b):
    x2 = x[0]
    adj2 = adj[0]
    b1r = b1.reshape(1, NHID)
    b2r = b2.reshape(1, 1)
    l1b = L1_b.reshape(NH, 1)
    l2w = L2_W.reshape(NH, 1)
    l2b = L2_b.reshape(1, 1)

    v = pl.pallas_call(
        _pass1_body,
        grid=(NB1,),
        in_specs=[
            pl.BlockSpec((BM, N), lambda i: (i, 0)),
            pl.BlockSpec((N, NFEAT), lambda i: (0, 0)),
            pl.BlockSpec((NFEAT, NHID), lambda i: (0, 0)),
            pl.BlockSpec((1, NHID), lambda i: (0, 0)),
            pl.BlockSpec((NHID, 1), lambda i: (0, 0)),
        ],
        out_specs=pl.BlockSpec((BM, 1), lambda i: (i, 0)),
        out_shape=jax.ShapeDtypeStruct((N, 1), jnp.float32),
        scratch_shapes=[pltpu.VMEM((N, NHID), jnp.float32)],
    )(adj2, x2, W1, b1r, W2)

    zp = _sc_pass2(adj2, v.reshape(N))

    z_tc = pl.pallas_call(
        _pass2_body,
        grid=(NB2,),
        in_specs=[
            pl.BlockSpec((BM, N), lambda i: (i, 0)),
            pl.BlockSpec((N, 1), lambda i: (0, 0)),
            pl.BlockSpec((1, 1), lambda i: (0, 0)),
        ],
        out_specs=pl.BlockSpec((BM, 1), lambda i: (i, 0)),
        out_shape=jax.ShapeDtypeStruct((N_TC, 1), jnp.float32),
    )(adj2, v, b2r)

    out = pl.pallas_call(
        _readout_body,
        grid=(NBK,),
        in_specs=[
            pl.BlockSpec((BK, N), lambda k: (k, 0)),
            pl.BlockSpec((N_TC, 1), lambda k: (0, 0)),
            pl.BlockSpec((R_SC, 16), lambda k: (0, 0)),
            pl.BlockSpec((BK, 1), lambda k: (k, 0)),
            pl.BlockSpec((BK, 1), lambda k: (k, 0)),
            pl.BlockSpec((1, 1), lambda k: (0, 0)),
            pl.BlockSpec((1, 1), lambda k: (0, 0)),
        ],
        out_specs=pl.BlockSpec((1, 1), lambda k: (0, 0)),
        out_shape=jax.ShapeDtypeStruct((1, 1), jnp.float32),
        scratch_shapes=[pltpu.VMEM((N, 1), jnp.float32)],
    )(L1_W, z_tc, zp.reshape(R_SC, 16), l1b, l2w, b2r, l2b)

    return out


# speedup vs baseline: 1.2880x; 1.2880x over previous
import functools

import jax
import jax.numpy as jnp
from jax import lax
from jax.experimental import pallas as pl
from jax.experimental.pallas import tpu as pltpu
from jax.experimental.pallas import tpu_sc as plsc

N = 10000
NFEAT = 128
NHID = 128
NH = N // 2

BM = 200
BK = 200

R_SC = 3200
N_TC = N - R_SC
RK_SC = 1600
NK_TC = NH - RK_SC

NW = 32
NCH = N // 16
UNROLL = 25

NB1 = N // BM
NB2 = N_TC // BM
NBK = NK_TC // BK


def _pass1_body(adj_ref, x_ref, w1_ref, b1_ref, w2_ref, v_ref, s1_ref):
    i = pl.program_id(0)

    @pl.when(i == 0)
    def _():
        s1_ref[...] = jnp.dot(x_ref[...], w1_ref[...],
                              preferred_element_type=jnp.float32)

    h = jnp.dot(adj_ref[...], s1_ref[...],
                preferred_element_type=jnp.float32)
    h = jnp.maximum(h + b1_ref[...], 0.0)
    v_ref[...] = jnp.dot(h, w2_ref[...],
                         preferred_element_type=jnp.float32)


_sc_mesh = plsc.VectorSubcoreMesh(core_axis_name="c", subcore_axis_name="s")


def _make_sc_matvec(row_lo, n_rows):
    rpw = n_rows // NW

    @functools.partial(
        pl.kernel,
        mesh=_sc_mesh,
        out_type=jax.ShapeDtypeStruct((n_rows * 16,), jnp.float32),
        scratch_types=[
            pltpu.VMEM((N,), jnp.float32),
            pltpu.VMEM((N,), jnp.float32),
            pltpu.VMEM((N,), jnp.float32),
            pltpu.VMEM((rpw * 16,), jnp.float32),
            pltpu.SemaphoreType.DMA,
            pltpu.SemaphoreType.DMA,
        ],
    )
    def sc_matvec(m_hbm, u_hbm, out_hbm, u_v, r0_v, r1_v, p_v, sem0, sem1):
        wid = lax.axis_index("s") * 2 + lax.axis_index("c")
        base = row_lo + wid * rpw
        pltpu.sync_copy(u_hbm, u_v)

        def row_dot(row_ref):
            def body(kk, accs):
                a = list(accs)
                for u in range(UNROLL):
                    j = kk * UNROLL + u
                    a[u % 5] = (a[u % 5]
                                + row_ref[pl.ds(j * 16, 16)]
                                * u_v[pl.ds(j * 16, 16)])
                return tuple(a)

            z5 = tuple(jnp.zeros((16,), jnp.float32) for _ in range(5))
            accs = lax.fori_loop(0, NCH // UNROLL, body, z5)
            return accs[0] + accs[1] + accs[2] + accs[3] + accs[4]

        pltpu.async_copy(m_hbm.at[base], r0_v, sem0)

        def pair(k, carry):
            r0 = base + 2 * k
            pltpu.async_copy(m_hbm.at[r0 + 1], r1_v, sem1)
            pltpu.make_async_copy(m_hbm.at[r0], r0_v, sem0).wait()
            p_v[pl.ds((2 * k) * 16, 16)] = row_dot(r0_v)
            nxt = jnp.minimum(r0 + 2, row_lo + n_rows - 1)
            pltpu.async_copy(m_hbm.at[nxt], r0_v, sem0)
            pltpu.make_async_copy(m_hbm.at[r0 + 1], r1_v, sem1).wait()
            p_v[pl.ds((2 * k + 1) * 16, 16)] = row_dot(r1_v)
            return carry

        lax.fori_loop(0, rpw // 2, pair, 0)
        pltpu.make_async_copy(m_hbm.at[base], r0_v, sem0).wait()
        pltpu.sync_copy(p_v, out_hbm.at[pl.ds(wid * rpw * 16, rpw * 16)])

    return sc_matvec


_sc_pass2 = _make_sc_matvec(N_TC, R_SC)
_sc_readout = _make_sc_matvec(NK_TC, RK_SC)


def _pass2_body(adj_ref, v_ref, b2_ref, z_ref):
    z_ref[...] = (jnp.dot(adj_ref[...], v_ref[...],
                          preferred_element_type=jnp.float32)
                  + b2_ref[...])


def _zasm_body(ztc_ref, zp_ref, b2_ref, z_ref):
    z_ref[:N_TC, :] = ztc_ref[...]
    z_ref[N_TC:, :] = (jnp.sum(zp_ref[...], axis=1, keepdims=True)
                       + b2_ref[...])


def _readout_body(l1w_ref, z_ref, l1b_ref, l2w_ref, o_ref):
    k = pl.program_id(0)
    h = jnp.dot(l1w_ref[...], z_ref[...],
                preferred_element_type=jnp.float32)
    h = jnp.maximum(h + l1b_ref[...], 0.0)
    part = jnp.sum(h * l2w_ref[...]).reshape(1, 1)

    @pl.when(k == 0)
    def _():
        o_ref[...] = part

    @pl.when(k > 0)
    def _():
        o_ref[...] += part


def _final_body(fp_ref, l1b_ref, l2w_ref, otc_ref, l2b_ref, o_ref):
    d = jnp.sum(fp_ref[...], axis=1, keepdims=True)
    h = jnp.maximum(d + l1b_ref[...], 0.0)
    o_ref[...] = (jnp.sum(h * l2w_ref[...]).reshape(1, 1)
                  + otc_ref[...] + l2b_ref[...])


def kernel(x, adj, W1, b1, W2, b2, L1_W, L1_b, L2_W, L2_b):
    x2 = x[0]
    adj2 = adj[0]
    b1r = b1.reshape(1, NHID)
    b2r = b2.reshape(1, 1)
    l1b = L1_b.reshape(NH, 1)
    l2w = L2_W.reshape(NH, 1)
    l2b = L2_b.reshape(1, 1)

    v = pl.pallas_call(
        _pass1_body,
        grid=(NB1,),
        in_specs=[
            pl.BlockSpec((BM, N), lambda i: (i, 0)),
            pl.BlockSpec((N, NFEAT), lambda i: (0, 0)),
            pl.BlockSpec((NFEAT, NHID), lambda i: (0, 0)),
            pl.BlockSpec((1, NHID), lambda i: (0, 0)),
            pl.BlockSpec((NHID, 1), lambda i: (0, 0)),
        ],
        out_specs=pl.BlockSpec((BM, 1), lambda i: (i, 0)),
        out_shape=jax.ShapeDtypeStruct((N, 1), jnp.float32),
        scratch_shapes=[pltpu.VMEM((N, NHID), jnp.float32)],
    )(adj2, x2, W1, b1r, W2)

    zp = _sc_pass2(adj2, v.reshape(N))

    z_tc = pl.pallas_call(
        _pass2_body,
        grid=(NB2,),
        in_specs=[
            pl.BlockSpec((BM, N), lambda i: (i, 0)),
            pl.BlockSpec((N, 1), lambda i: (0, 0)),
            pl.BlockSpec((1, 1), lambda i: (0, 0)),
        ],
        out_specs=pl.BlockSpec((BM, 1), lambda i: (i, 0)),
        out_shape=jax.ShapeDtypeStruct((N_TC, 1), jnp.float32),
    )(adj2, v, b2r)

    z = pl.pallas_call(
        _zasm_body,
        in_specs=[
            pl.BlockSpec((N_TC, 1), lambda: (0, 0)),
            pl.BlockSpec((R_SC, 16), lambda: (0, 0)),
            pl.BlockSpec((1, 1), lambda: (0, 0)),
        ],
        out_specs=pl.BlockSpec((N, 1), lambda: (0, 0)),
        out_shape=jax.ShapeDtypeStruct((N, 1), jnp.float32),
    )(z_tc, zp.reshape(R_SC, 16), b2r)

    fp = _sc_readout(L1_W, z.reshape(N))

    out_tc = pl.pallas_call(
        _readout_body,
        grid=(NBK,),
        in_specs=[
            pl.BlockSpec((BK, N), lambda k: (k, 0)),
            pl.BlockSpec((N, 1), lambda k: (0, 0)),
            pl.BlockSpec((BK, 1), lambda k: (k, 0)),
            pl.BlockSpec((BK, 1), lambda k: (k, 0)),
        ],
        out_specs=pl.BlockSpec((1, 1), lambda k: (0, 0)),
        out_shape=jax.ShapeDtypeStruct((1, 1), jnp.float32),
    )(L1_W, z, l1b, l2w)

    out = pl.pallas_call(
        _final_body,
        in_specs=[
            pl.BlockSpec((RK_SC, 16), lambda: (0, 0)),
            pl.BlockSpec((RK_SC, 1), lambda: (0, 0)),
            pl.BlockSpec((RK_SC, 1), lambda: (0, 0)),
            pl.BlockSpec((1, 1), lambda: (0, 0)),
            pl.BlockSpec((1, 1), lambda: (0, 0)),
        ],
        out_specs=pl.BlockSpec((1, 1), lambda: (0, 0)),
        out_shape=jax.ShapeDtypeStruct((1, 1), jnp.float32),
    )(fp.reshape(RK_SC, 16), l1b[NK_TC:], l2w[NK_TC:], out_tc, l2b)

    return out
